# baseline (speedup 1.0000x reference)
import functools

import jax
import jax.numpy as jnp
from jax import lax
from jax.experimental import pallas as pl
from jax.experimental.pallas import tpu as pltpu
from jax.experimental.pallas import tpu_sc as plsc

_B, _C, _H, _W, _N = 64, 1, 256, 256, 256
_NC, _NS, _L = 2, 16, 16
_NW = _NC * _NS
_RPW = _B // _NW
_NCHUNK = _N // _L
_NT = 5
_QH = _H // 4
_NQ = _RPW * _NT * 4
_RING = 7


def _pull_body(t1, t2, t3, t4, t5, i1, i2, i3, i4, i5, mask_hbm, out_hbm,
               buf_v, idx_v, m_v, acc_v,
               sem0, sem1, sem2, sem3, sem4, sem5, sem6, semi):
    tags = (t1, t2, t3, t4, t5)
    inds = (i1, i2, i3, i4, i5)
    sems = (sem0, sem1, sem2, sem3, sem4, sem5, sem6)
    wid = lax.axis_index("s") * _NC + lax.axis_index("c")
    rows = [wid * _RPW + rb for rb in range(_RPW)]

    def fire_quarter(g):
        rb, rest = divmod(g, _NT * 4)
        t, qt = divmod(rest, 4)
        return pltpu.async_copy(
            tags[t].at[rows[rb], pl.ds(qt * _QH, _QH)],
            buf_v.at[g % _RING], sems[g % _RING])

    def fire_small(rb):
        cps = [pltpu.async_copy(
            mask_hbm.at[pl.ds(rows[rb], 1)], m_v.at[rb], semi)]
        for t in range(_NT):
            cps.append(pltpu.async_copy(
                inds[t].at[pl.ds(rows[rb], 1)], idx_v.at[rb, t], semi))
        return cps

    def row_scale(rb):
        num_vec = jnp.zeros((_L,), jnp.int32)
        for k in range(_NCHUNK):
            mb = m_v[rb, 0, pl.ds(k * _L, _L)] > 0
            num_vec = num_vec + plsc.all_reduce_population_count(mb)
        return 1.0 / (num_vec.astype(jnp.float32) + 1e-4)

    small = fire_small(0)
    pend = {g: fire_quarter(g) for g in range(_RING)}
    for cp in small:
        cp.wait()
    scale = [row_scale(0), None]

    acc = jnp.zeros((_L,), jnp.float32)
    s1 = [jnp.zeros((_L,), jnp.float32) for _ in range(_NCHUNK)]
    s2 = [jnp.zeros((_L,), jnp.float32) for _ in range(_NCHUNK)]
    for j in range(_RPW * _NT):
        rb, t = divmod(j, _NT)
        if j == _NT - 1:
            small = fire_small(1)
        for q in range(4):
            pend.pop(4 * j + q).wait()
        for g in range(4 * j + 4, 4 * j + 7):
            if j > 0 and g < _NQ:
                pend[g] = fire_quarter(g)
        if j == _NT:
            for cp in small:
                cp.wait()
            scale[1] = row_scale(1)
        c = (4 * j) % _RING
        for k in range(_NCHUNK):
            ind = idx_v[rb, t, 0, pl.ds(k * _L, _L)]
            ih = lax.shift_right_logical(ind, 8)
            iw = lax.bitwise_and(ind, 255)
            cq = lax.shift_right_logical(ih, 6) + c
            slot = jnp.where(cq >= _RING, cq - _RING, cq)
            ihq = lax.bitwise_and(ih, _QH - 1)
            g = plsc.load_gather(buf_v, [slot, ihq, iw])
            s1[k] = s1[k] + g
            s2[k] = s2[k] + g * g
        if 4 * j + 7 < _NQ:
            pend[4 * j + 7] = fire_quarter(4 * j + 7)
        if t == _NT - 1:
            for k in range(_NCHUNK):
                ssd = s2[k] - s1[k] * s1[k] * 0.2
                mf = m_v[rb, 0, pl.ds(k * _L, _L)].astype(jnp.float32)
                acc = acc + (mf * ssd) * scale[rb]
                s1[k] = jnp.zeros((_L,), jnp.float32)
                s2[k] = jnp.zeros((_L,), jnp.float32)

    acc_v[...] = acc
    pltpu.sync_copy(acc_v, out_hbm.at[wid])


@jax.jit
def _ae_pull(t1, t2, t3, t4, t5, i1, i2, i3, i4, i5, mask_i32):
    mesh = plsc.VectorSubcoreMesh(core_axis_name="c", subcore_axis_name="s")
    run = functools.partial(
        pl.kernel,
        mesh=mesh,
        compiler_params=pltpu.CompilerParams(
            needs_layout_passes=False, use_tc_tiling_on_sc=True),
        out_type=jax.ShapeDtypeStruct((_NW, _L), jnp.float32),
        scratch_types=[
            pltpu.VMEM((_RING, _QH, _W), jnp.float32),
            pltpu.VMEM((_RPW, _NT, 1, _N), jnp.int32),
            pltpu.VMEM((_RPW, 1, _N), jnp.int32),
            pltpu.VMEM((_L,), jnp.float32),
        ] + [pltpu.SemaphoreType.DMA] * (_RING + 1),
    )(_pull_body)
    return run(t1, t2, t3, t4, t5, i1, i2, i3, i4, i5, mask_i32)


def kernel(tag1, tag2, tag3, tag4, tag5, ind1, ind2, ind3, ind4, ind5, mask):
    t3d = [t.reshape(_B, _H, _W) for t in (tag1, tag2, tag3, tag4, tag5)]
    mask_i32 = mask.astype(jnp.int32)
    partials = _ae_pull(*t3d, ind1, ind2, ind3, ind4, ind5, mask_i32)
    pull = jnp.sum(partials)
    push = jnp.zeros((), jnp.float32)
    return pull, push


# speedup vs baseline: 1.0230x; 1.0230x over previous
import functools

import jax
import jax.numpy as jnp
from jax import lax
from jax.experimental import pallas as pl
from jax.experimental.pallas import tpu as pltpu
from jax.experimental.pallas import tpu_sc as plsc

_B, _C, _H, _W, _N = 64, 1, 256, 256, 256
_NC, _NS, _L = 2, 16, 16
_NW = _NC * _NS
_RPW = _B // _NW
_NCHUNK = _N // _L
_NT = 5
_HH = _H // 2
_NHALF = _RPW * _NT * 2
_RING = 3


def _pull_body(t1, t2, t3, t4, t5, i1, i2, i3, i4, i5, mask_hbm, out_hbm,
               buf_v, idx_v, m_v, s1_v, s2_v, acc_v,
               sem0, sem1, sem2, semi):
    tags = (t1, t2, t3, t4, t5)
    inds = (i1, i2, i3, i4, i5)
    sems = (sem0, sem1, sem2)
    wid = lax.axis_index("s") * _NC + lax.axis_index("c")
    rows = [wid * _RPW + rb for rb in range(_RPW)]

    def fire_half(g):
        rb, rest = divmod(g, _NT * 2)
        t, hf = divmod(rest, 2)
        return pltpu.async_copy(
            tags[t].at[rows[rb], pl.ds(hf * _HH, _HH)],
            buf_v.at[g % _RING], sems[g % _RING])

    def fire_small(rb):
        cps = [pltpu.async_copy(
            mask_hbm.at[pl.ds(rows[rb], 1)], m_v.at[rb], semi)]
        for t in range(_NT):
            cps.append(pltpu.async_copy(
                inds[t].at[pl.ds(rows[rb], 1)], idx_v.at[rb, t], semi))
        return cps

    def row_scale(rb):
        def body(k, num_vec):
            mb = m_v[rb, 0, pl.ds(k * _L, _L)] > 0
            return num_vec + plsc.all_reduce_population_count(mb)
        num_vec = lax.fori_loop(0, _NCHUNK, body, jnp.zeros((_L,), jnp.int32))
        return 1.0 / (num_vec.astype(jnp.float32) + 1e-4)

    small = fire_small(0)
    pend = {g: fire_half(g) for g in range(_RING)}
    for cp in small:
        cp.wait()
    scale = [row_scale(0), None]

    acc = jnp.zeros((_L,), jnp.float32)
    for j in range(_RPW * _NT):
        rb, t = divmod(j, _NT)
        if j == _NT - 1:
            small = fire_small(1)
        g_lo, g_hi = 2 * j, 2 * j + 1
        pend.pop(g_lo).wait()
        pend.pop(g_hi).wait()
        if j > 0 and g_lo + 2 < _NHALF:
            pend[g_lo + 2] = fire_half(g_lo + 2)
        if j == _NT:
            for cp in small:
                cp.wait()
            scale[1] = row_scale(1)
        s_lo, s_hi = g_lo % _RING, g_hi % _RING

        def chunk_body(k, carry, rb=rb, t=t, s_lo=s_lo, s_hi=s_hi):
            off = k * _L
            ind = idx_v[rb, t, 0, pl.ds(off, _L)]
            ih = lax.shift_right_logical(ind, 8)
            iw = lax.bitwise_and(ind, 255)
            ihm = lax.bitwise_and(ih, _HH - 1)
            v_lo = plsc.load_gather(buf_v.at[s_lo], [ihm, iw])
            v_hi = plsc.load_gather(buf_v.at[s_hi], [ihm, iw])
            g = jnp.where(ih < _HH, v_lo, v_hi)
            if t == 0:
                s1_v[0, pl.ds(off, _L)] = g
                s2_v[0, pl.ds(off, _L)] = g * g
            else:
                s1_v[0, pl.ds(off, _L)] = s1_v[0, pl.ds(off, _L)] + g
                s2_v[0, pl.ds(off, _L)] = s2_v[0, pl.ds(off, _L)] + g * g
            return carry

        lax.fori_loop(0, _NCHUNK, chunk_body, 0)
        if g_hi + 2 < _NHALF:
            pend[g_hi + 2] = fire_half(g_hi + 2)
        if t == _NT - 1:
            def fold_body(k, a, rb=rb, sc=scale[rb]):
                off = k * _L
                s1 = s1_v[0, pl.ds(off, _L)]
                s2 = s2_v[0, pl.ds(off, _L)]
                ssd = s2 - s1 * s1 * 0.2
                mf = m_v[rb, 0, pl.ds(off, _L)].astype(jnp.float32)
                return a + (mf * ssd) * sc

            acc = lax.fori_loop(0, _NCHUNK, fold_body, acc)

    acc_v[...] = acc
    pltpu.sync_copy(acc_v, out_hbm.at[wid])


@jax.jit
def _ae_pull(t1, t2, t3, t4, t5, i1, i2, i3, i4, i5, mask_i32):
    mesh = plsc.VectorSubcoreMesh(core_axis_name="c", subcore_axis_name="s")
    run = functools.partial(
        pl.kernel,
        mesh=mesh,
        compiler_params=pltpu.CompilerParams(
            needs_layout_passes=False, use_tc_tiling_on_sc=True),
        out_type=jax.ShapeDtypeStruct((_NW, _L), jnp.float32),
        scratch_types=[
            pltpu.VMEM((_RING, _HH, _W), jnp.float32),
            pltpu.VMEM((_RPW, _NT, 1, _N), jnp.int32),
            pltpu.VMEM((_RPW, 1, _N), jnp.int32),
            pltpu.VMEM((1, _N), jnp.float32),
            pltpu.VMEM((1, _N), jnp.float32),
            pltpu.VMEM((_L,), jnp.float32),
        ] + [pltpu.SemaphoreType.DMA] * (_RING + 1),
    )(_pull_body)
    return run(t1, t2, t3, t4, t5, i1, i2, i3, i4, i5, mask_i32)


def kernel(tag1, tag2, tag3, tag4, tag5, ind1, ind2, ind3, ind4, ind5, mask):
    t3d = [t.reshape(_B, _H, _W) for t in (tag1, tag2, tag3, tag4, tag5)]
    mask_i32 = mask.astype(jnp.int32)
    partials = _ae_pull(*t3d, ind1, ind2, ind3, ind4, ind5, mask_i32)
    pull = jnp.sum(partials)
    push = jnp.zeros((), jnp.float32)
    return pull, push


# speedup vs baseline: 1.0312x; 1.0080x over previous
import functools

import jax
import jax.numpy as jnp
from jax import lax
from jax.experimental import pallas as pl
from jax.experimental.pallas import tpu as pltpu
from jax.experimental.pallas import tpu_sc as plsc

_B, _C, _H, _W, _N = 64, 1, 256, 256, 256
_NC, _NS, _L = 2, 16, 16
_NW = _NC * _NS
_RPW = _B // _NW
_NCHUNK = _N // _L
_NT = 5


def _pull_body(t1, t2, t3, t4, t5, i1, i2, i3, i4, i5, mask_hbm, out_hbm,
               tile_v, idx_v, m_v, acc_v, sem, semi):
    tags = (t1, t2, t3, t4, t5)
    inds = (i1, i2, i3, i4, i5)
    wid = lax.axis_index("s") * _NC + lax.axis_index("c")

    acc = jnp.zeros((_L,), jnp.float32)
    for rb in range(_RPW):
        b = wid * _RPW + rb
        small = [pltpu.async_copy(mask_hbm.at[pl.ds(b, 1)], m_v, semi)]
        for t in range(_NT):
            small.append(pltpu.async_copy(
                inds[t].at[pl.ds(b, 1)], idx_v.at[t], semi))
        pltpu.async_copy(tags[0].at[b], tile_v, sem).wait()
        for cp in small:
            cp.wait()

        num_vec = jnp.zeros((_L,), jnp.int32)
        for k in range(_NCHUNK):
            mb = m_v[0, pl.ds(k * _L, _L)] > 0
            num_vec = num_vec + plsc.all_reduce_population_count(mb)
        scale = 1.0 / (num_vec.astype(jnp.float32) + 1e-4)

        s1 = [jnp.zeros((_L,), jnp.float32) for _ in range(_NCHUNK)]
        s2 = [jnp.zeros((_L,), jnp.float32) for _ in range(_NCHUNK)]
        for t in range(_NT):
            for k in range(_NCHUNK):
                ind = idx_v[t, 0, pl.ds(k * _L, _L)]
                ih = lax.shift_right_logical(ind, 8)
                iw = lax.bitwise_and(ind, 255)
                g = plsc.load_gather(tile_v, [ih, iw])
                s1[k] = s1[k] + g
                s2[k] = s2[k] + g * g
            if t + 1 < _NT:
                pltpu.sync_copy(tags[t + 1].at[b], tile_v)

        for k in range(_NCHUNK):
            ssd = s2[k] - s1[k] * s1[k] * 0.2
            mf = m_v[0, pl.ds(k * _L, _L)].astype(jnp.float32)
            acc = acc + (mf * ssd) * scale

    acc_v[...] = acc
    pltpu.sync_copy(acc_v, out_hbm.at[wid])


@jax.jit
def _ae_pull(t1, t2, t3, t4, t5, i1, i2, i3, i4, i5, mask_i32):
    mesh = plsc.VectorSubcoreMesh(core_axis_name="c", subcore_axis_name="s")
    run = functools.partial(
        pl.kernel,
        mesh=mesh,
        compiler_params=pltpu.CompilerParams(
            needs_layout_passes=False, use_tc_tiling_on_sc=True),
        out_type=jax.ShapeDtypeStruct((_NW, _L), jnp.float32),
        scratch_types=[
            pltpu.VMEM((_H, _W), jnp.float32),
            pltpu.VMEM((_NT, 1, _N), jnp.int32),
            pltpu.VMEM((1, _N), jnp.int32),
            pltpu.VMEM((_L,), jnp.float32),
            pltpu.SemaphoreType.DMA,
            pltpu.SemaphoreType.DMA,
        ],
    )(_pull_body)
    return run(t1, t2, t3, t4, t5, i1, i2, i3, i4, i5, mask_i32)


def kernel(tag1, tag2, tag3, tag4, tag5, ind1, ind2, ind3, ind4, ind5, mask):
    t3d = [t.reshape(_B, _H, _W) for t in (tag1, tag2, tag3, tag4, tag5)]
    mask_i32 = mask.astype(jnp.int32)
    partials = _ae_pull(*t3d, ind1, ind2, ind3, ind4, ind5, mask_i32)
    pull = jnp.sum(partials)
    push = jnp.zeros((), jnp.float32)
    return pull, push
